# baseline (speedup 1.0000x reference)
import functools

import jax
import jax.numpy as jnp
import numpy as np
from jax import lax
from jax.experimental import pallas as pl
from jax.experimental.pallas import tpu as pltpu
from jax.experimental.pallas import tpu_sc as plsc

NUM_CORES = 2
NUM_SUBCORES = 16
NW = NUM_CORES * NUM_SUBCORES
LANES = 16
BATCH = 16384
FACTORS = 64
B_PER_W = BATCH // NW
NBLK = B_PER_W // LANES
PASS_ROWS = 256
NPASS = B_PER_W // PASS_ROWS
NSEM = 8
ROWS_PER_SEM = PASS_ROWS // NSEM


def _body(users_hbm, items_hbm, gu_hbm, gi_hbm, bi_hbm,
          xui_out, beta_out, gu_out, gi_out,
          uidx_v, iidx_v, fu, fi, bv, xui_v, sems, semb):
  wid = lax.axis_index("s") * NUM_CORES + lax.axis_index("c")
  base = wid * B_PER_W

  pltpu.sync_copy(users_hbm.at[pl.ds(wid * NBLK, NBLK)], uidx_v)
  pltpu.sync_copy(items_hbm.at[pl.ds(wid * NBLK, NBLK)], iidx_v)

  bcopies = [
      pltpu.async_copy(bi_hbm.at[iidx_v.at[b]],
                       bv.at[pl.ds(b * LANES, LANES)], semb)
      for b in range(NBLK)
  ]
  for c in bcopies:
    c.wait()

  lane = lax.iota(jnp.int32, LANES)

  for p in range(NPASS):
    for s in range(NSEM):
      def fire(k, _, s=s):
        b = p * (PASS_ROWS // LANES) + s * (ROWS_PER_SEM // LANES) + k
        r0 = (s * ROWS_PER_SEM + k * LANES)
        uvec = uidx_v[b, pl.ds(0, LANES)]
        ivec = iidx_v[b, pl.ds(0, LANES)]
        for t in range(LANES):
          u = lax.squeeze(lax.slice(uvec, (t,), (t + 1,)), (0,))
          i = lax.squeeze(lax.slice(ivec, (t,), (t + 1,)), (0,))
          pltpu.async_copy(gu_hbm.at[pl.ds(u, 1)],
                           fu.at[pl.ds(r0 + t, 1)], sems.at[s])
          pltpu.async_copy(gi_hbm.at[pl.ds(i, 1)],
                           fi.at[pl.ds(r0 + t, 1)], sems.at[s])
        return 0

      lax.fori_loop(0, ROWS_PER_SEM // LANES, fire, 0)

    for s in range(NSEM):
      dst = pl.ds(s * ROWS_PER_SEM, ROWS_PER_SEM)
      pltpu.make_async_copy(
          gu_hbm.at[pl.ds(0, ROWS_PER_SEM)], fu.at[dst], sems.at[s]).wait()
      pltpu.make_async_copy(
          gi_hbm.at[pl.ds(0, ROWS_PER_SEM)], fi.at[dst], sems.at[s]).wait()

    def group(g, _):
      res = jnp.zeros((LANES,), jnp.float32)
      for t in range(LANES):
        r = g * LANES + t
        acc = fu[r, pl.ds(0, LANES)] * fi[r, pl.ds(0, LANES)]
        for c in range(1, FACTORS // LANES):
          acc += (fu[r, pl.ds(c * LANES, LANES)] *
                  fi[r, pl.ds(c * LANES, LANES)])
        res = jnp.where(lane == t, jnp.sum(acc), res)
      xui_v[pl.ds(p * PASS_ROWS + g * LANES, LANES)] = (
          res + bv[pl.ds(p * PASS_ROWS + g * LANES, LANES)])
      return 0

    lax.fori_loop(0, PASS_ROWS // LANES, group, 0)

    pltpu.sync_copy(fu, gu_out.at[pl.ds(base + p * PASS_ROWS, PASS_ROWS)])
    pltpu.sync_copy(fi, gi_out.at[pl.ds(base + p * PASS_ROWS, PASS_ROWS)])

  pltpu.sync_copy(bv, beta_out.at[pl.ds(base, B_PER_W)])
  pltpu.sync_copy(xui_v, xui_out.at[pl.ds(base, B_PER_W)])


@jax.jit
def _run(users2, items2, Gu, Gi, bi_flat):
  mesh = plsc.VectorSubcoreMesh(core_axis_name="c", subcore_axis_name="s")
  f = pl.kernel(
      _body,
      out_type=(
          jax.ShapeDtypeStruct((BATCH,), jnp.float32),
          jax.ShapeDtypeStruct((BATCH,), jnp.float32),
          jax.ShapeDtypeStruct((BATCH, FACTORS), jnp.float32),
          jax.ShapeDtypeStruct((BATCH, FACTORS), jnp.float32),
      ),
      mesh=mesh,
      compiler_params=pltpu.CompilerParams(needs_layout_passes=False),
      scratch_types=[
          pltpu.VMEM((NBLK, LANES), jnp.int32),
          pltpu.VMEM((NBLK, LANES), jnp.int32),
          pltpu.VMEM((PASS_ROWS, FACTORS), jnp.float32),
          pltpu.VMEM((PASS_ROWS, FACTORS), jnp.float32),
          pltpu.VMEM((B_PER_W,), jnp.float32),
          pltpu.VMEM((B_PER_W,), jnp.float32),
          pltpu.SemaphoreType.DMA((NSEM,)),
          pltpu.SemaphoreType.DMA,
      ],
  )
  return f(users2, items2, Gu, Gi, bi_flat)


def kernel(users_indices, items_indices, Gu, Gi, Bi):
  users2 = users_indices.astype(jnp.int32).reshape(BATCH // LANES, LANES)
  items2 = items_indices.astype(jnp.int32).reshape(BATCH // LANES, LANES)
  bi_flat = Bi.reshape(Bi.shape[0])
  xui, beta_i, gamma_u, gamma_i = _run(users2, items2, Gu, Gi, bi_flat)
  return (xui, beta_i, gamma_u, gamma_i)
